# baseline (speedup 1.0000x reference)
import functools

import jax
import jax.numpy as jnp
from jax import lax
from jax.experimental import pallas as pl
from jax.experimental.pallas import tpu as pltpu
from jax.experimental.pallas import tpu_sc as plsc

LANES = 16


def kernel(h, r, t, E, R):
    B = h.shape[0]
    V, D = E.shape
    NR = R.shape[0]
    mesh = plsc.VectorSubcoreMesh(core_axis_name="c", subcore_axis_name="s")
    NW = mesh.num_cores * mesh.num_subcores
    b_per_w = B // NW
    n_groups = b_per_w // LANES

    Et = E.T
    Rf = R.T.reshape(NR * D)

    @functools.partial(
        pl.kernel,
        out_type=jax.ShapeDtypeStruct((B,), jnp.float32),
        mesh=mesh,
        scratch_types=[
            pltpu.VMEM((b_per_w,), jnp.int32),
            pltpu.VMEM((b_per_w,), jnp.int32),
            pltpu.VMEM((b_per_w,), jnp.int32),
            pltpu.VMEM((D, b_per_w), jnp.float32),
            pltpu.VMEM((D, b_per_w), jnp.float32),
            pltpu.VMEM((NR * D,), jnp.float32),
            pltpu.VMEM((b_per_w,), jnp.float32),
            pltpu.SemaphoreType.DMA,
        ],
        compiler_params=pltpu.CompilerParams(
            needs_layout_passes=False, use_tc_tiling_on_sc=False),
    )
    def transe(h_hbm, r_hbm, t_hbm, E_hbm, R_hbm, out_hbm,
               h_v, t_v, r_v, eh_v, et_v, R_v, out_v, sem):
        wid = lax.axis_index("s") * mesh.num_cores + lax.axis_index("c")
        base = wid * b_per_w

        pltpu.sync_copy(h_hbm.at[pl.ds(base, b_per_w)], h_v)
        pltpu.sync_copy(t_hbm.at[pl.ds(base, b_per_w)], t_v)

        copies = []
        for d in range(D):
            copies.append(pltpu.async_copy(
                E_hbm.at[d].at[h_v], eh_v.at[d], sem))
            copies.append(pltpu.async_copy(
                E_hbm.at[d].at[t_v], et_v.at[d], sem))

        pltpu.sync_copy(r_hbm.at[pl.ds(base, b_per_w)], r_v)
        pltpu.sync_copy(R_hbm, R_v)
        for cp in copies:
            cp.wait()

        def group(g, _):
            rg = r_v[pl.ds(g * LANES, LANES)]

            def body(d, carry):
                acc, ridx = carry
                a = eh_v[d, pl.ds(g * LANES, LANES)]
                c = et_v[d, pl.ds(g * LANES, LANES)]
                b = plsc.load_gather(R_v, [ridx])
                return acc + jnp.abs(a + b - c), ridx + NR

            acc, _ = lax.fori_loop(
                0, D, body, (jnp.zeros((LANES,), jnp.float32), rg))
            out_v[pl.ds(g * LANES, LANES)] = acc
            return 0

        lax.fori_loop(0, n_groups, group, 0)

        pltpu.sync_copy(out_v, out_hbm.at[pl.ds(base, b_per_w)])

    return transe(h, r, t, Et, Rf)


# speedup vs baseline: 5.0363x; 5.0363x over previous
import functools

import jax
import jax.numpy as jnp
from jax import lax
from jax.experimental import pallas as pl
from jax.experimental.pallas import tpu as pltpu
from jax.experimental.pallas import tpu_sc as plsc

LANES = 16


def kernel(h, r, t, E, R):
    B = h.shape[0]
    V, D = E.shape
    mesh = plsc.VectorSubcoreMesh(core_axis_name="c", subcore_axis_name="s")
    NW = mesh.num_cores * mesh.num_subcores
    b_per_w = B // NW

    @functools.partial(
        pl.kernel,
        out_type=jax.ShapeDtypeStruct((B,), jnp.float32),
        mesh=mesh,
        scratch_types=[
            pltpu.VMEM((b_per_w,), jnp.int32),
            pltpu.VMEM((b_per_w,), jnp.int32),
            pltpu.VMEM((b_per_w,), jnp.int32),
            pltpu.VMEM((b_per_w, D), jnp.float32),
            pltpu.VMEM((b_per_w, D), jnp.float32),
            pltpu.VMEM((b_per_w, D), jnp.float32),
            pltpu.VMEM((b_per_w,), jnp.float32),
            pltpu.SemaphoreType.DMA,
        ],
        compiler_params=pltpu.CompilerParams(
            needs_layout_passes=False, use_tc_tiling_on_sc=False),
    )
    def transe(h_hbm, r_hbm, t_hbm, E_hbm, R_hbm, out_hbm,
               h_v, t_v, r_v, eh_v, et_v, rr_v, out_v, sem):
        wid = lax.axis_index("s") * mesh.num_cores + lax.axis_index("c")
        base = wid * b_per_w

        pltpu.sync_copy(h_hbm.at[pl.ds(base, b_per_w)], h_v)
        pltpu.sync_copy(t_hbm.at[pl.ds(base, b_per_w)], t_v)
        pltpu.sync_copy(r_hbm.at[pl.ds(base, b_per_w)], r_v)

        c1 = pltpu.async_copy(E_hbm.at[h_v], eh_v, sem)
        c2 = pltpu.async_copy(E_hbm.at[t_v], et_v, sem)
        c3 = pltpu.async_copy(R_hbm.at[r_v], rr_v, sem)
        c1.wait()
        c2.wait()
        c3.wait()

        lanes = lax.iota(jnp.int32, LANES)

        def body(g, _):
            acc = jnp.zeros((LANES,), jnp.float32)
            for j in range(LANES):
                i = g * LANES + j
                lo = jnp.abs(eh_v[i, pl.ds(0, LANES)]
                             + rr_v[i, pl.ds(0, LANES)]
                             - et_v[i, pl.ds(0, LANES)])
                hi = jnp.abs(eh_v[i, pl.ds(LANES, LANES)]
                             + rr_v[i, pl.ds(LANES, LANES)]
                             - et_v[i, pl.ds(LANES, LANES)])
                acc = jnp.where(lanes == j, jnp.sum(lo + hi), acc)
            out_v[pl.ds(g * LANES, LANES)] = acc
            return 0

        lax.fori_loop(0, b_per_w // LANES, body, 0)

        pltpu.sync_copy(out_v, out_hbm.at[pl.ds(base, b_per_w)])

    return transe(h, r, t, E, R)
